# baseline (speedup 1.0000x reference)
import functools
import math

import jax
import jax.numpy as jnp
from jax import lax
from jax.experimental import pallas as pl
from jax.experimental.pallas import tpu as pltpu
from jax.experimental.pallas import tpu_sc as plsc

_SMOOTH = 0.1


@functools.cache
def _sc_gather_contrib(N, V):
    info = plsc.get_sparse_core_info()
    nc, ns, nl = info.num_cores, info.num_subcores, info.num_lanes
    nw = nc * ns
    per_w = N // nw
    eps = _SMOOTH / (V - 2)
    cconst = eps * (V - 2) * math.log(eps) + (1.0 - _SMOOTH) * math.log(1.0 - _SMOOTH)
    mesh = plsc.VectorSubcoreMesh(core_axis_name="c", subcore_axis_name="s")

    @functools.partial(
        pl.kernel,
        mesh=mesh,
        out_type=jax.ShapeDtypeStruct((nw * nl,), jnp.float32),
        scratch_types=[
            pltpu.SMEM((per_w,), jnp.int32),
            pltpu.VMEM((per_w,), jnp.int32),
            pltpu.VMEM((per_w, 8, 128), jnp.float32),
            pltpu.VMEM((per_w, 128), jnp.float32),
            pltpu.VMEM((nl,), jnp.float32),
            pltpu.SemaphoreType.DMA,
        ],
    )
    def sc_kernel(x2d, yh, outh, y_s, y_vm, g_v, z_v, c_v, sem):
        wid = lax.axis_index("s") * nc + lax.axis_index("c")
        base = wid * per_w
        pltpu.sync_copy(yh.at[pl.ds(base, per_w)], y_vm)
        for c in range(per_w // nl):
            yv16 = y_vm[pl.ds(c * nl, nl)]
            for j in range(nl):
                y_s[c * nl + j] = yv16[j]
        pltpu.sync_copy(x2d.at[pl.ds(base, per_w), pl.ds(0, 128)], z_v)
        for r in range(per_w):
            col = pl.multiple_of((y_s[r] // 128) * 128, 128)
            row0 = base + (r // 8) * 8
            pltpu.sync_copy(x2d.at[pl.ds(row0, 8), pl.ds(col, 128)], g_v.at[r])
        lanes = lax.iota(jnp.int32, nl)
        one = jnp.full((nl,), 1, jnp.int32)
        m0 = (one - jnp.minimum(lanes, one)).astype(jnp.float32)
        cvec0 = m0 * cconst
        acc = jnp.zeros((nl,), jnp.float32)
        for r in range(per_w):
            y_r = y_s[r]
            yv = jnp.full((nl,), y_r, jnp.int32)
            lc = pl.multiple_of(((y_r % 128) // nl) * nl, nl)
            gv = g_v[r, r % 8, pl.ds(lc, nl)]
            zv = z_v[r, pl.ds(0, nl)]
            gm = (one - jnp.minimum(jnp.abs(lanes - yv % nl), one)).astype(
                jnp.float32
            )
            pm = jnp.minimum(yv, one).astype(jnp.float32)
            row_vec = (
                gv * gm * (eps - (1.0 - _SMOOTH)) + zv * (m0 * eps) + cvec0
            )
            acc = acc + row_vec * pm
        c_v[...] = acc
        pltpu.sync_copy(c_v, outh.at[pl.ds(wid * nl, nl)])

    return sc_kernel


@functools.cache
def _tc_loss(N, V, br):
    nblk = N // br // 2

    def body(x0_ref, x1_ref, y0_ref, y1_ref, out_ref, acc_ref):
        pid = pl.program_id(0)

        @pl.when(pid == 0)
        def _():
            acc_ref[0] = 0.0

        s0 = jnp.sum(x0_ref[...], axis=1, keepdims=True)
        s0 = jnp.where(y0_ref[...] != 0, s0, 0.0)
        s1 = jnp.sum(x1_ref[...], axis=1, keepdims=True)
        s1 = jnp.where(y1_ref[...] != 0, s1, 0.0)
        acc_ref[0] += jnp.sum(s0) + jnp.sum(s1)

        @pl.when(pid == nblk - 1)
        def _():
            out_ref[0, 0] = acc_ref[0]

    return pl.pallas_call(
        body,
        grid=(nblk,),
        in_specs=[
            pl.BlockSpec((br, V), lambda i: (i, 0)),
            pl.BlockSpec((br, V), lambda i: (i + nblk, 0)),
            pl.BlockSpec((br, 1), lambda i: (i, 0)),
            pl.BlockSpec((br, 1), lambda i: (i + nblk, 0)),
        ],
        out_specs=pl.BlockSpec((1, 1), lambda i: (0, 0), memory_space=pltpu.SMEM),
        out_shape=jax.ShapeDtypeStruct((1, 1), jnp.float32),
        scratch_shapes=[pltpu.SMEM((1,), jnp.float32)],
    )


def kernel(x, y, norm):
    V = x.shape[-1]
    x2 = x.reshape(-1, V)
    N = x2.shape[0]
    yf = y.reshape(-1).astype(jnp.int32)
    contrib = _sc_gather_contrib(N, V)(x2, yf)
    y2 = yf.reshape(N, 1)
    masked_total = _tc_loss(N, V, 32)(x2, x2, y2, y2)[0, 0]
    eps = _SMOOTH / (V - 2)
    return (jnp.sum(contrib) - eps * masked_total) / norm


# speedup vs baseline: 1.0044x; 1.0044x over previous
import functools
import math

import jax
import jax.numpy as jnp
from jax import lax
from jax.experimental import pallas as pl
from jax.experimental.pallas import tpu as pltpu
from jax.experimental.pallas import tpu_sc as plsc

_SMOOTH = 0.1


@functools.cache
def _sc_gather_contrib(N, V):
    info = plsc.get_sparse_core_info()
    nc, ns, nl = info.num_cores, info.num_subcores, info.num_lanes
    nw = nc * ns
    per_w = N // nw
    eps = _SMOOTH / (V - 2)
    cconst = eps * (V - 2) * math.log(eps) + (1.0 - _SMOOTH) * math.log(1.0 - _SMOOTH)
    mesh = plsc.VectorSubcoreMesh(core_axis_name="c", subcore_axis_name="s")

    @functools.partial(
        pl.kernel,
        mesh=mesh,
        out_type=jax.ShapeDtypeStruct((nw * nl,), jnp.float32),
        scratch_types=[
            pltpu.SMEM((per_w,), jnp.int32),
            pltpu.VMEM((per_w,), jnp.int32),
            pltpu.VMEM((per_w, 8, 128), jnp.float32),
            pltpu.VMEM((per_w, 128), jnp.float32),
            pltpu.VMEM((nl,), jnp.float32),
            pltpu.SemaphoreType.DMA,
        ],
    )
    def sc_kernel(x2d, yh, outh, y_s, y_vm, g_v, z_v, c_v, sem):
        wid = lax.axis_index("s") * nc + lax.axis_index("c")
        base = wid * per_w
        pltpu.sync_copy(yh.at[pl.ds(base, per_w)], y_vm)
        for c in range(per_w // nl):
            yv16 = y_vm[pl.ds(c * nl, nl)]
            for j in range(nl):
                y_s[c * nl + j] = yv16[j]
        pltpu.sync_copy(x2d.at[pl.ds(base, per_w), pl.ds(0, 128)], z_v)
        for r in range(per_w):
            col = pl.multiple_of((y_s[r] // 128) * 128, 128)
            row0 = base + (r // 8) * 8
            pltpu.sync_copy(x2d.at[pl.ds(row0, 8), pl.ds(col, 128)], g_v.at[r])
        lanes = lax.iota(jnp.int32, nl)
        one = jnp.full((nl,), 1, jnp.int32)
        m0 = (one - jnp.minimum(lanes, one)).astype(jnp.float32)
        cvec0 = m0 * cconst
        acc = jnp.zeros((nl,), jnp.float32)
        for r in range(per_w):
            y_r = y_s[r]
            yv = jnp.full((nl,), y_r, jnp.int32)
            lc = pl.multiple_of(((y_r % 128) // nl) * nl, nl)
            gv = g_v[r, r % 8, pl.ds(lc, nl)]
            zv = z_v[r, pl.ds(0, nl)]
            gm = (one - jnp.minimum(jnp.abs(lanes - yv % nl), one)).astype(
                jnp.float32
            )
            pm = jnp.minimum(yv, one).astype(jnp.float32)
            row_vec = (
                gv * gm * (eps - (1.0 - _SMOOTH)) + zv * (m0 * eps) + cvec0
            )
            acc = acc + row_vec * pm
        c_v[...] = acc
        pltpu.sync_copy(c_v, outh.at[pl.ds(wid * nl, nl)])

    return sc_kernel


@functools.cache
def _tc_loss(N, V, br):
    nblk = N // br
    vmain = (V // 128) * 128
    vtail = V - vmain

    def body(xm_ref, xt_ref, y_ref, out_ref, acc_ref):
        pid = pl.program_id(0)

        @pl.when(pid == 0)
        def _():
            acc_ref[0] = 0.0

        srow = jnp.sum(xm_ref[...], axis=1, keepdims=True)
        tmask = lax.broadcasted_iota(jnp.int32, (br, 128), 1) < vtail
        srow = srow + jnp.sum(
            jnp.where(tmask, xt_ref[...], 0.0), axis=1, keepdims=True
        )
        srow = jnp.where(y_ref[...] != 0, srow, 0.0)
        acc_ref[0] += jnp.sum(srow)

        @pl.when(pid == nblk - 1)
        def _():
            out_ref[0, 0] = acc_ref[0]

    return pl.pallas_call(
        body,
        grid=(nblk,),
        in_specs=[
            pl.BlockSpec((br, vmain), lambda i: (i, 0)),
            pl.BlockSpec((br, 128), lambda i: (i, vmain // 128)),
            pl.BlockSpec((br, 1), lambda i: (i, 0)),
        ],
        out_specs=pl.BlockSpec((1, 1), lambda i: (0, 0), memory_space=pltpu.SMEM),
        out_shape=jax.ShapeDtypeStruct((1, 1), jnp.float32),
        scratch_shapes=[pltpu.SMEM((1,), jnp.float32)],
    )


def kernel(x, y, norm):
    V = x.shape[-1]
    x2 = x.reshape(-1, V)
    N = x2.shape[0]
    yf = y.reshape(-1).astype(jnp.int32)
    contrib = _sc_gather_contrib(N, V)(x2, yf)
    y2 = yf.reshape(N, 1)
    masked_total = _tc_loss(N, V, 64)(x2, x2, y2)[0, 0]
    eps = _SMOOTH / (V - 2)
    return (jnp.sum(contrib) - eps * masked_total) / norm
